# baseline (speedup 1.0000x reference)
import functools

import jax
import jax.numpy as jnp
from jax import lax
from jax.experimental import pallas as pl
from jax.experimental.pallas import tpu as pltpu
from jax.experimental.pallas import tpu_sc as plsc

N = 10000
E = 160000
D = 128
H = 128
HP = 144
OUT = 313

_BM = 640

_NC = 2
_NS = 16
_NW = _NC * _NS
_L = 16
_CHUNK = 128
_NCHUNK = E // _CHUNK
_JMAX = (_NCHUNK + _NW - 1) // _NW


def _sc_mesh():
    return plsc.VectorSubcoreMesh(core_axis_name="c", subcore_axis_name="s")


def _gather_pair_sum(tab, src, dst):
    @functools.partial(
        pl.kernel,
        out_type=jax.ShapeDtypeStruct((E, D), jnp.float32),
        mesh=_sc_mesh(),
        scratch_types=[
            pltpu.VMEM((_CHUNK,), jnp.int32),
            pltpu.VMEM((_CHUNK,), jnp.int32),
            pltpu.VMEM((_CHUNK, D), jnp.float32),
            pltpu.VMEM((_CHUNK, D), jnp.float32),
            pltpu.SemaphoreType.DMA,
            pltpu.SemaphoreType.DMA,
        ],
    )
    def body(tab_hbm, src_hbm, dst_hbm, out_hbm, sidx_v, didx_v, ra_v, rb_v,
             sem_a, sem_b):
        w = lax.axis_index("s") * _NC + lax.axis_index("c")

        def step(j, carry):
            c = w + _NW * j

            @pl.when(c < _NCHUNK)
            def _():
                base = c * _CHUNK
                pltpu.sync_copy(src_hbm.at[pl.ds(base, _CHUNK)], sidx_v)
                pltpu.sync_copy(dst_hbm.at[pl.ds(base, _CHUNK)], didx_v)
                cpa = pltpu.async_copy(tab_hbm.at[sidx_v], ra_v, sem_a)
                cpb = pltpu.async_copy(tab_hbm.at[didx_v], rb_v, sem_b)
                cpa.wait()
                cpb.wait()

                def add_row(r, cc):
                    for k in range(D // _L):
                        sl = pl.ds(k * _L, _L)
                        ra_v[r, sl] = ra_v[r, sl] + rb_v[r, sl]
                    return cc

                lax.fori_loop(0, _CHUNK, add_row, 0)
                pltpu.sync_copy(ra_v, out_hbm.at[pl.ds(base, _CHUNK)])

            return carry

        lax.fori_loop(0, _JMAX, step, 0)

    return body(tab, src, dst)


_N16 = 10240
_NA = 10240


def _loop_attr_pass(src, dst, ea0, ea1):
    cols = _N16 // _NS

    @functools.partial(
        pl.kernel,
        out_type=(jax.ShapeDtypeStruct((_NC, 3 * _N16), jnp.float32),
                  jax.ShapeDtypeStruct((_NC, _NS, 3 * _N16), jnp.float32)),
        mesh=_sc_mesh(),
        scratch_types=[
            pltpu.VMEM((3 * _N16,), jnp.float32),
            pltpu.VMEM((_CHUNK,), jnp.int32),
            pltpu.VMEM((_CHUNK,), jnp.int32),
            pltpu.VMEM((_CHUNK,), jnp.float32),
            pltpu.VMEM((_CHUNK,), jnp.float32),
        ],
        compiler_params=pltpu.CompilerParams(needs_layout_passes=False),
    )
    def body(src_hbm, dst_hbm, ea0_hbm, ea1_hbm, out_hbm, scr_hbm, tab_v,
             sidx_v, didx_v, a0_v, a1_v):
        cid = lax.axis_index("c")
        sid = lax.axis_index("s")
        w = sid * _NC + cid

        def ztab(k, cc):
            tab_v[pl.ds(k * _L, _L)] = jnp.zeros((_L,), jnp.float32)
            return cc

        lax.fori_loop(0, 3 * _N16 // _L, ztab, 0)

        def step(j, carry):
            c = w + _NW * j

            @pl.when(c < _NCHUNK)
            def _():
                base = c * _CHUNK
                pltpu.sync_copy(src_hbm.at[pl.ds(base, _CHUNK)], sidx_v)
                pltpu.sync_copy(dst_hbm.at[pl.ds(base, _CHUNK)], didx_v)
                pltpu.sync_copy(ea0_hbm.at[pl.ds(base, _CHUNK)], a0_v)
                pltpu.sync_copy(ea1_hbm.at[pl.ds(base, _CHUNK)], a1_v)
                for g in range(_CHUNK // _L):
                    sl = pl.ds(g * _L, _L)
                    si = sidx_v[sl]
                    di = didx_v[sl]
                    m = (si != di).astype(jnp.float32)
                    plsc.addupdate_scatter(tab_v, [di], m)
                    plsc.addupdate_scatter(tab_v, [di + _N16], m * a0_v[sl])
                    plsc.addupdate_scatter(tab_v, [di + 2 * _N16],
                                           m * a1_v[sl])

            return carry

        lax.fori_loop(0, _JMAX, step, 0)

        pltpu.sync_copy(tab_v, scr_hbm.at[cid, sid])
        plsc.subcore_barrier()
        for q in range(3):
            for t in range(_NS):
                pltpu.sync_copy(
                    scr_hbm.at[cid, t, pl.ds(q * _N16 + sid * cols, cols)],
                    tab_v.at[pl.ds(t * cols, cols)])

            def qred(k, cc):
                sl = pl.ds(k * _L, _L)
                v = tab_v[sl]
                for t in range(1, _NS):
                    v = v + tab_v[pl.ds(t * cols + k * _L, _L)]
                tab_v[sl] = v
                return cc

            lax.fori_loop(0, cols // _L, qred, 0)
            pltpu.sync_copy(tab_v.at[pl.ds(0, cols)],
                            out_hbm.at[cid, pl.ds(q * _N16 + sid * cols,
                                                  cols)])

    return body(src, dst, ea0, ea1)


def _gat_alpha_pass(asn, adn, src, dst, ea0, ea1, cvec):
    cols = _N16 // _NS

    @functools.partial(
        pl.kernel,
        out_type=(jax.ShapeDtypeStruct((E,), jnp.float32),
                  jax.ShapeDtypeStruct((_NC, _N16), jnp.float32),
                  jax.ShapeDtypeStruct((_NC, _NS, _N16), jnp.float32)),
        mesh=_sc_mesh(),
        scratch_types=[
            pltpu.VMEM((N,), jnp.float32),
            pltpu.VMEM((N,), jnp.float32),
            pltpu.VMEM((_N16,), jnp.float32),
            pltpu.VMEM((_CHUNK,), jnp.int32),
            pltpu.VMEM((_CHUNK,), jnp.int32),
            pltpu.VMEM((_CHUNK,), jnp.float32),
            pltpu.VMEM((_CHUNK,), jnp.float32),
            pltpu.VMEM((_CHUNK,), jnp.float32),
            pltpu.VMEM((16,), jnp.float32),
        ],
        compiler_params=pltpu.CompilerParams(needs_layout_passes=False),
    )
    def body(asn_hbm, adn_hbm, src_hbm, dst_hbm, ea0_hbm, ea1_hbm, cvec_hbm,
             ex_out, den_out, den_scr, asn_v, adn_v, den_v, sidx_v, didx_v,
             a0_v, a1_v, ex_v, cv_v):
        cid = lax.axis_index("c")
        sid = lax.axis_index("s")
        w = sid * _NC + cid

        pltpu.sync_copy(asn_hbm, asn_v)
        pltpu.sync_copy(adn_hbm, adn_v)
        pltpu.sync_copy(cvec_hbm, cv_v)

        def zden(k, cc):
            den_v[pl.ds(k * _L, _L)] = jnp.zeros((_L,), jnp.float32)
            return cc

        lax.fori_loop(0, _N16 // _L, zden, 0)
        cvv = cv_v[pl.ds(0, _L)]
        c0 = cvv[0]
        c1 = cvv[1]

        def step(j, carry):
            c = w + _NW * j

            @pl.when(c < _NCHUNK)
            def _():
                base = c * _CHUNK
                pltpu.sync_copy(src_hbm.at[pl.ds(base, _CHUNK)], sidx_v)
                pltpu.sync_copy(dst_hbm.at[pl.ds(base, _CHUNK)], didx_v)
                pltpu.sync_copy(ea0_hbm.at[pl.ds(base, _CHUNK)], a0_v)
                pltpu.sync_copy(ea1_hbm.at[pl.ds(base, _CHUNK)], a1_v)
                for g in range(_CHUNK // _L):
                    sl = pl.ds(g * _L, _L)
                    si = sidx_v[sl]
                    di = didx_v[sl]
                    av = plsc.load_gather(asn_v, [si])
                    bv = plsc.load_gather(adn_v, [di])
                    al = av + bv + a0_v[sl] * c0 + a1_v[sl] * c1
                    al = jnp.where(al >= 0, al, 0.2 * al)
                    exg = jnp.where(si != di, jnp.exp(al), 0.0)
                    ex_v[sl] = exg
                    plsc.addupdate_scatter(den_v, [di], exg)
                pltpu.sync_copy(ex_v, ex_out.at[pl.ds(base, _CHUNK)])

            return carry

        lax.fori_loop(0, _JMAX, step, 0)

        pltpu.sync_copy(den_v, den_scr.at[cid, sid])
        plsc.subcore_barrier()
        for t in range(_NS):
            pltpu.sync_copy(den_scr.at[cid, t, pl.ds(sid * cols, cols)],
                            den_v.at[pl.ds(t * cols, cols)])

        def dred(k, cc):
            sl = pl.ds(k * _L, _L)
            v = den_v[sl]
            for t in range(1, _NS):
                v = v + den_v[pl.ds(t * cols + k * _L, _L)]
            den_v[sl] = v
            return cc

        lax.fori_loop(0, cols // _L, dred, 0)
        pltpu.sync_copy(den_v.at[pl.ds(0, cols)],
                        den_out.at[cid, pl.ds(sid * cols, cols)])

    return body(asn, adn, src, dst, ea0, ea1, cvec)


def _gat_scatter_pass(h, src, dst, ex):
    rpt = _NA // _NS

    @functools.partial(
        pl.kernel,
        out_type=jax.ShapeDtypeStruct((_NC, _NA, D), jnp.float32),
        mesh=_sc_mesh(),
        scratch_types=[
            pltpu.VMEM((_CHUNK,), jnp.int32),
            pltpu.VMEM((_CHUNK,), jnp.int32),
            pltpu.VMEM((_CHUNK,), jnp.float32),
            pltpu.VMEM((_CHUNK, D), jnp.float32),
            pltpu.VMEM_SHARED((_NA, D), jnp.float32),
            pltpu.SemaphoreType.DMA,
        ],
    )
    def body(h_hbm, src_hbm, dst_hbm, ex_hbm, num_out, sidx_v, didx_v, ex_v,
             hrows_v, acc_sh, sem):
        cid = lax.axis_index("c")
        sid = lax.axis_index("s")
        w = sid * _NC + cid

        def zrow(r, cc):
            for k in range(D // _L):
                hrows_v[r, pl.ds(k * _L, _L)] = jnp.zeros((_L,), jnp.float32)
            return cc

        lax.fori_loop(0, _CHUNK, zrow, 0)
        for b in range(rpt // _CHUNK):
            pltpu.sync_copy(hrows_v,
                            acc_sh.at[pl.ds(sid * rpt + b * _CHUNK, _CHUNK)])
        plsc.subcore_barrier()

        def step(j, carry):
            c = w + _NW * j

            @pl.when(c < _NCHUNK)
            def _():
                base = c * _CHUNK
                pltpu.sync_copy(src_hbm.at[pl.ds(base, _CHUNK)], sidx_v)
                pltpu.sync_copy(dst_hbm.at[pl.ds(base, _CHUNK)], didx_v)
                pltpu.sync_copy(ex_hbm.at[pl.ds(base, _CHUNK)], ex_v)
                cp = pltpu.async_copy(h_hbm.at[sidx_v], hrows_v, sem)
                cp.wait()

                def scale_group(g, cc):
                    sl = pl.ds(g * _L, _L)
                    exg = ex_v[sl]
                    for i in range(_L):
                        r = g * _L + i
                        exb = jnp.full((_L,), exg[i], jnp.float32)
                        for k in range(D // _L):
                            ksl = pl.ds(k * _L, _L)
                            hrows_v[r, ksl] = hrows_v[r, ksl] * exb
                    return cc

                lax.fori_loop(0, _CHUNK // _L, scale_group, 0)
                pltpu.sync_copy(hrows_v, acc_sh.at[didx_v], add=True)

            return carry

        lax.fori_loop(0, _JMAX, step, 0)
        plsc.subcore_barrier()
        pltpu.sync_copy(acc_sh.at[pl.ds(sid * rpt, rpt)],
                        num_out.at[cid, pl.ds(sid * rpt, rpt)])

    return body(h, src, dst, ex)


_BN = 1000


def _prep_body(x_ref, w_ref, a2_ref, c_ref, cnt_ref, lsum_ref,
               h_ref, sd_ref, exl_ref, la_ref):
    h = jnp.dot(x_ref[...], w_ref[...], preferred_element_type=jnp.float32)
    sd = jnp.dot(h, a2_ref[...], preferred_element_type=jnp.float32)
    la = lsum_ref[...] / jnp.maximum(cnt_ref[...], 1.0)
    ael = jnp.dot(la, c_ref[...], preferred_element_type=jnp.float32)
    al = sd[:, 0:1] + sd[:, 1:2] + ael
    al = jnp.where(al >= 0, al, 0.2 * al)
    h_ref[...] = h
    sd_ref[...] = sd
    exl_ref[...] = jnp.exp(al)
    la_ref[...] = la


def _prep_layer(x, W, a_s, a_d, c, cnt_col, lsum):
    a2 = jnp.stack([a_s, a_d], axis=1)
    ccol = c[:, None]
    grid = (N // _BN,)
    return pl.pallas_call(
        _prep_body,
        grid=grid,
        in_specs=[
            pl.BlockSpec((_BN, D), lambda i: (i, 0)),
            pl.BlockSpec((D, D), lambda i: (0, 0)),
            pl.BlockSpec((D, 2), lambda i: (0, 0)),
            pl.BlockSpec((2, 1), lambda i: (0, 0)),
            pl.BlockSpec((_BN, 1), lambda i: (i, 0)),
            pl.BlockSpec((_BN, 2), lambda i: (i, 0)),
        ],
        out_specs=[
            pl.BlockSpec((_BN, D), lambda i: (i, 0)),
            pl.BlockSpec((_BN, 2), lambda i: (i, 0)),
            pl.BlockSpec((_BN, 1), lambda i: (i, 0)),
            pl.BlockSpec((_BN, 2), lambda i: (i, 0)),
        ],
        out_shape=[
            jax.ShapeDtypeStruct((N, D), jnp.float32),
            jax.ShapeDtypeStruct((N, 2), jnp.float32),
            jax.ShapeDtypeStruct((N, 1), jnp.float32),
            jax.ShapeDtypeStruct((N, 2), jnp.float32),
        ],
    )(x, W, a2, ccol, cnt_col, lsum)


def _finish_body(p_ref, dsum_ref, exl_ref, h_ref, b_ref, y_ref):
    num = p_ref[0] + p_ref[1] + exl_ref[...] * h_ref[...]
    den = dsum_ref[...] + exl_ref[...] + 1e-16
    y_ref[...] = jnp.maximum(num / den + b_ref[...], 0.0)


def _finish_layer(nump, dsum_col, exl, h, b):
    grid = (N // _BN,)
    return pl.pallas_call(
        _finish_body,
        grid=grid,
        in_specs=[
            pl.BlockSpec((2, _BN, D), lambda i: (0, i, 0)),
            pl.BlockSpec((_BN, 1), lambda i: (i, 0)),
            pl.BlockSpec((_BN, 1), lambda i: (i, 0)),
            pl.BlockSpec((_BN, D), lambda i: (i, 0)),
            pl.BlockSpec((1, D), lambda i: (0, 0)),
        ],
        out_specs=pl.BlockSpec((_BN, D), lambda i: (i, 0)),
        out_shape=jax.ShapeDtypeStruct((N, D), jnp.float32),
    )(nump, dsum_col, exl, h, b[None, :])


def _finish_body3(p_ref, dsum_ref, exl_ref, h_ref, b_ref, y0_ref,
                  wl2_ref, bl2_ref, y4_ref):
    num = p_ref[0] + p_ref[1] + exl_ref[...] * h_ref[...]
    den = dsum_ref[...] + exl_ref[...] + 1e-16
    y1 = jnp.maximum(num / den + b_ref[...], 0.0)
    s = y0_ref[...] + y1
    y4 = jnp.dot(s, wl2_ref[...], preferred_element_type=jnp.float32)
    y4_ref[...] = jnp.maximum(y4 + bl2_ref[...], 0.0)


def _finish_layer2(nump, dsum_col, exl, h, b, y0, Wl2, bl2):
    grid = (N // _BN,)
    return pl.pallas_call(
        _finish_body3,
        grid=grid,
        in_specs=[
            pl.BlockSpec((2, _BN, D), lambda i: (0, i, 0)),
            pl.BlockSpec((_BN, 1), lambda i: (i, 0)),
            pl.BlockSpec((_BN, 1), lambda i: (i, 0)),
            pl.BlockSpec((_BN, D), lambda i: (i, 0)),
            pl.BlockSpec((1, D), lambda i: (0, 0)),
            pl.BlockSpec((_BN, D), lambda i: (i, 0)),
            pl.BlockSpec((D, D), lambda i: (0, 0)),
            pl.BlockSpec((1, D), lambda i: (0, 0)),
        ],
        out_specs=pl.BlockSpec((_BN, D), lambda i: (i, 0)),
        out_shape=jax.ShapeDtypeStruct((N, D), jnp.float32),
    )(nump, dsum_col, exl, h, b[None, :], y0, Wl2, bl2[None, :])


def _edge_mlp_body(q_ref, ea_ref, wl3a_ref, bl3_ref, wl3b_ref, wm1_ref, bm1_ref,
                   wm2_ref, bm2_ref, wl4_ref, bl4_ref, out_ref):
    za = jnp.dot(q_ref[...], wl3a_ref[...], preferred_element_type=jnp.float32)
    eb = jnp.dot(ea_ref[...], wl3b_ref[...], preferred_element_type=jnp.float32)
    u0 = jnp.maximum(za + eb + bl3_ref[...], 0.0)
    u1 = jnp.dot(u0, wm1_ref[...], preferred_element_type=jnp.float32)
    u1 = jnp.maximum(u1 + bm1_ref[...], 0.0)
    u2 = jnp.dot(u1, wm2_ref[...], preferred_element_type=jnp.float32)
    u2 = jnp.maximum(u2 + bm2_ref[...], 0.0)
    yb = jnp.dot(u2, wl4_ref[...], preferred_element_type=jnp.float32)
    out_ref[...] = yb + bl4_ref[...]


def _edge_mlp(q, ea8, wl3a, bl3p, wl3b8, wm1p, bm1p, wm2p, bm2p, wl4p, bl4p):
    grid = (E // _BM,)
    return pl.pallas_call(
        _edge_mlp_body,
        grid=grid,
        in_specs=[
            pl.BlockSpec((_BM, D), lambda i: (i, 0)),
            pl.BlockSpec((_BM, 8), lambda i: (i, 0)),
            pl.BlockSpec((D, HP), lambda i: (0, 0)),
            pl.BlockSpec((1, HP), lambda i: (0, 0)),
            pl.BlockSpec((8, HP), lambda i: (0, 0)),
            pl.BlockSpec((HP, HP), lambda i: (0, 0)),
            pl.BlockSpec((1, HP), lambda i: (0, 0)),
            pl.BlockSpec((HP, HP), lambda i: (0, 0)),
            pl.BlockSpec((1, HP), lambda i: (0, 0)),
            pl.BlockSpec((HP, OUT), lambda i: (0, 0)),
            pl.BlockSpec((1, OUT), lambda i: (0, 0)),
        ],
        out_specs=pl.BlockSpec((_BM, OUT), lambda i: (i, 0)),
        out_shape=jax.ShapeDtypeStruct((E, OUT), jnp.float32),
    )(q, ea8, wl3a, bl3p, wl3b8, wm1p, bm1p, wm2p, bm2p, wl4p, bl4p)


def _pad2(a, r, c):
    return jnp.pad(a, ((0, r - a.shape[0]), (0, c - a.shape[1])))


def kernel(x, edge_index, edge_attr, shift, W1, a1_src, a1_dst, We1, a1_edge, b1,
           W2, a2_src, a2_dst, We2, a2_edge, b2, W_l2, b_l2, W_l3, b_l3,
           Wm1, bm1, Wm2, bm2, W_l4, b_l4):
    src = edge_index[0]
    dst = edge_index[1]
    ea0 = edge_attr[:, 0]
    ea1 = edge_attr[:, 1]
    la, _ = _loop_attr_pass(src, dst, ea0, ea1)
    las = la[0] + la[1]
    cnt_col = las[:N, None]
    lsum = jnp.stack([las[_N16:_N16 + N], las[2 * _N16:2 * _N16 + N]], axis=1)

    def gat_core(xin, W, a_s, a_d, We, a_e):
        c = We @ a_e
        h, sd, exl, _ = _prep_layer(xin, W, a_s, a_d, c, cnt_col, lsum)
        asn = sd[:, 0]
        adn = sd[:, 1]
        cvec = jnp.pad(c, (0, 14))
        ex, denp, _ = _gat_alpha_pass(asn, adn, src, dst, ea0, ea1, cvec)
        nump = _gat_scatter_pass(h, src, dst, ex)
        dsum_col = (denp[0, :N] + denp[1, :N])[:, None]
        return nump, dsum_col, exl, h

    nump1, dsum1, exl1, h1 = gat_core(x, W1, a1_src, a1_dst, We1, a1_edge)
    y0 = _finish_layer(nump1, dsum1, exl1, h1, b1)
    nump2, dsum2, exl2, h2 = gat_core(y0, W2, a2_src, a2_dst, We2, a2_edge)
    y4 = _finish_layer2(nump2, dsum2, exl2, h2, b2, y0, W_l2, b_l2)

    q = _gather_pair_sum(y4, src, dst)

    ea8 = jnp.pad(edge_attr, ((0, 0), (0, 6)))
    wl3a = jnp.pad(W_l3[:H], ((0, 0), (0, HP - (H + 2))))
    bl3p = jnp.pad(b_l3, (0, HP - (H + 2)))[None, :]
    wl3b8 = jnp.pad(W_l3[H:], ((0, 6), (0, HP - (H + 2))))
    wm1p = _pad2(Wm1, HP, HP)
    wm2p = _pad2(Wm2, HP, HP)
    wl4p = jnp.pad(W_l4, ((0, HP - (H + 2)), (0, 0)))
    bm1p = jnp.pad(bm1, (0, HP - (H + 2)))[None, :]
    bm2p = jnp.pad(bm2, (0, HP - (H + 2)))[None, :]
    bl4p = b_l4[None, :]

    return _edge_mlp(q, ea8, wl3a, bl3p, wl3b8, wm1p, bm1p, wm2p, bm2p, wl4p,
                     bl4p)


# speedup vs baseline: 1.0637x; 1.0637x over previous
import functools

import jax
import jax.numpy as jnp
from jax import lax
from jax.experimental import pallas as pl
from jax.experimental.pallas import tpu as pltpu
from jax.experimental.pallas import tpu_sc as plsc

N = 10000
E = 160000
D = 128
H = 128
HP = 144
OUT = 313

_BM = 640

_NC = 2
_NS = 16
_NW = _NC * _NS
_L = 16
_CHUNK = 128
_NCHUNK = E // _CHUNK
_JMAX = (_NCHUNK + _NW - 1) // _NW


def _sc_mesh():
    return plsc.VectorSubcoreMesh(core_axis_name="c", subcore_axis_name="s")


def _gather_pair_sum(tab, src, dst):
    @functools.partial(
        pl.kernel,
        out_type=jax.ShapeDtypeStruct((E, D), jnp.float32),
        mesh=_sc_mesh(),
        scratch_types=[
            pltpu.VMEM((_CHUNK,), jnp.int32),
            pltpu.VMEM((_CHUNK,), jnp.int32),
            pltpu.VMEM((_CHUNK, D), jnp.float32),
            pltpu.VMEM((_CHUNK, D), jnp.float32),
            pltpu.SemaphoreType.DMA,
            pltpu.SemaphoreType.DMA,
        ],
    )
    def body(tab_hbm, src_hbm, dst_hbm, out_hbm, sidx_v, didx_v, ra_v, rb_v,
             sem_a, sem_b):
        w = lax.axis_index("s") * _NC + lax.axis_index("c")

        def step(j, carry):
            c = w + _NW * j

            @pl.when(c < _NCHUNK)
            def _():
                base = c * _CHUNK
                pltpu.sync_copy(src_hbm.at[pl.ds(base, _CHUNK)], sidx_v)
                pltpu.sync_copy(dst_hbm.at[pl.ds(base, _CHUNK)], didx_v)
                cpa = pltpu.async_copy(tab_hbm.at[sidx_v], ra_v, sem_a)
                cpb = pltpu.async_copy(tab_hbm.at[didx_v], rb_v, sem_b)
                cpa.wait()
                cpb.wait()

                def add_row(r, cc):
                    for k in range(D // _L):
                        sl = pl.ds(k * _L, _L)
                        ra_v[r, sl] = ra_v[r, sl] + rb_v[r, sl]
                    return cc

                lax.fori_loop(0, _CHUNK, add_row, 0)
                pltpu.sync_copy(ra_v, out_hbm.at[pl.ds(base, _CHUNK)])

            return carry

        lax.fori_loop(0, _JMAX, step, 0)

    return body(tab, src, dst)


_N16 = 10240
_NA = 10240


def _loop_attr_pass(src, dst, ea0, ea1):
    cols = _N16 // _NS

    @functools.partial(
        pl.kernel,
        out_type=(jax.ShapeDtypeStruct((_NC, 3 * _N16), jnp.float32),
                  jax.ShapeDtypeStruct((_NC, _NS, 3 * _N16), jnp.float32)),
        mesh=_sc_mesh(),
        scratch_types=[
            pltpu.VMEM((3 * _N16,), jnp.float32),
            pltpu.VMEM((_CHUNK,), jnp.int32),
            pltpu.VMEM((_CHUNK,), jnp.int32),
            pltpu.VMEM((_CHUNK,), jnp.float32),
            pltpu.VMEM((_CHUNK,), jnp.float32),
        ],
        compiler_params=pltpu.CompilerParams(needs_layout_passes=False),
    )
    def body(src_hbm, dst_hbm, ea0_hbm, ea1_hbm, out_hbm, scr_hbm, tab_v,
             sidx_v, didx_v, a0_v, a1_v):
        cid = lax.axis_index("c")
        sid = lax.axis_index("s")
        w = sid * _NC + cid

        def ztab(k, cc):
            tab_v[pl.ds(k * _L, _L)] = jnp.zeros((_L,), jnp.float32)
            return cc

        lax.fori_loop(0, 3 * _N16 // _L, ztab, 0)

        def step(j, carry):
            c = w + _NW * j

            @pl.when(c < _NCHUNK)
            def _():
                base = c * _CHUNK
                pltpu.sync_copy(src_hbm.at[pl.ds(base, _CHUNK)], sidx_v)
                pltpu.sync_copy(dst_hbm.at[pl.ds(base, _CHUNK)], didx_v)
                pltpu.sync_copy(ea0_hbm.at[pl.ds(base, _CHUNK)], a0_v)
                pltpu.sync_copy(ea1_hbm.at[pl.ds(base, _CHUNK)], a1_v)
                for g in range(_CHUNK // _L):
                    sl = pl.ds(g * _L, _L)
                    si = sidx_v[sl]
                    di = didx_v[sl]
                    m = (si != di).astype(jnp.float32)
                    plsc.addupdate_scatter(tab_v, [di], m)
                    plsc.addupdate_scatter(tab_v, [di + _N16], m * a0_v[sl])
                    plsc.addupdate_scatter(tab_v, [di + 2 * _N16],
                                           m * a1_v[sl])

            return carry

        lax.fori_loop(0, _JMAX, step, 0)

        pltpu.sync_copy(tab_v, scr_hbm.at[cid, sid])
        plsc.subcore_barrier()
        for q in range(3):
            for t in range(_NS):
                pltpu.sync_copy(
                    scr_hbm.at[cid, t, pl.ds(q * _N16 + sid * cols, cols)],
                    tab_v.at[pl.ds(t * cols, cols)])

            def qred(k, cc):
                sl = pl.ds(k * _L, _L)
                v = tab_v[sl]
                for t in range(1, _NS):
                    v = v + tab_v[pl.ds(t * cols + k * _L, _L)]
                tab_v[sl] = v
                return cc

            lax.fori_loop(0, cols // _L, qred, 0)
            pltpu.sync_copy(tab_v.at[pl.ds(0, cols)],
                            out_hbm.at[cid, pl.ds(q * _N16 + sid * cols,
                                                  cols)])

    return body(src, dst, ea0, ea1)


def _gat_alpha_pass(asn, adn, src, dst, ea0, ea1, cvec):
    cols = _N16 // _NS

    @functools.partial(
        pl.kernel,
        out_type=(jax.ShapeDtypeStruct((E,), jnp.float32),
                  jax.ShapeDtypeStruct((_NC, _N16), jnp.float32),
                  jax.ShapeDtypeStruct((_NC, _NS, _N16), jnp.float32)),
        mesh=_sc_mesh(),
        scratch_types=[
            pltpu.VMEM((N,), jnp.float32),
            pltpu.VMEM((N,), jnp.float32),
            pltpu.VMEM((_N16,), jnp.float32),
            pltpu.VMEM((_CHUNK,), jnp.int32),
            pltpu.VMEM((_CHUNK,), jnp.int32),
            pltpu.VMEM((_CHUNK,), jnp.float32),
            pltpu.VMEM((_CHUNK,), jnp.float32),
            pltpu.VMEM((_CHUNK,), jnp.float32),
            pltpu.VMEM((16,), jnp.float32),
        ],
        compiler_params=pltpu.CompilerParams(needs_layout_passes=False),
    )
    def body(asn_hbm, adn_hbm, src_hbm, dst_hbm, ea0_hbm, ea1_hbm, cvec_hbm,
             ex_out, den_out, den_scr, asn_v, adn_v, den_v, sidx_v, didx_v,
             a0_v, a1_v, ex_v, cv_v):
        cid = lax.axis_index("c")
        sid = lax.axis_index("s")
        w = sid * _NC + cid

        pltpu.sync_copy(asn_hbm, asn_v)
        pltpu.sync_copy(adn_hbm, adn_v)
        pltpu.sync_copy(cvec_hbm, cv_v)

        def zden(k, cc):
            den_v[pl.ds(k * _L, _L)] = jnp.zeros((_L,), jnp.float32)
            return cc

        lax.fori_loop(0, _N16 // _L, zden, 0)
        cvv = cv_v[pl.ds(0, _L)]
        c0 = cvv[0]
        c1 = cvv[1]

        def step(j, carry):
            c = w + _NW * j

            @pl.when(c < _NCHUNK)
            def _():
                base = c * _CHUNK
                pltpu.sync_copy(src_hbm.at[pl.ds(base, _CHUNK)], sidx_v)
                pltpu.sync_copy(dst_hbm.at[pl.ds(base, _CHUNK)], didx_v)
                pltpu.sync_copy(ea0_hbm.at[pl.ds(base, _CHUNK)], a0_v)
                pltpu.sync_copy(ea1_hbm.at[pl.ds(base, _CHUNK)], a1_v)
                for g in range(_CHUNK // _L):
                    sl = pl.ds(g * _L, _L)
                    si = sidx_v[sl]
                    di = didx_v[sl]
                    av = plsc.load_gather(asn_v, [si])
                    bv = plsc.load_gather(adn_v, [di])
                    al = av + bv + a0_v[sl] * c0 + a1_v[sl] * c1
                    al = jnp.where(al >= 0, al, 0.2 * al)
                    exg = jnp.where(si != di, jnp.exp(al), 0.0)
                    ex_v[sl] = exg
                    plsc.addupdate_scatter(den_v, [di], exg)
                pltpu.sync_copy(ex_v, ex_out.at[pl.ds(base, _CHUNK)])

            return carry

        lax.fori_loop(0, _JMAX, step, 0)

        pltpu.sync_copy(den_v, den_scr.at[cid, sid])
        plsc.subcore_barrier()
        for t in range(_NS):
            pltpu.sync_copy(den_scr.at[cid, t, pl.ds(sid * cols, cols)],
                            den_v.at[pl.ds(t * cols, cols)])

        def dred(k, cc):
            sl = pl.ds(k * _L, _L)
            v = den_v[sl]
            for t in range(1, _NS):
                v = v + den_v[pl.ds(t * cols + k * _L, _L)]
            den_v[sl] = v
            return cc

        lax.fori_loop(0, cols // _L, dred, 0)
        pltpu.sync_copy(den_v.at[pl.ds(0, cols)],
                        den_out.at[cid, pl.ds(sid * cols, cols)])

    return body(asn, adn, src, dst, ea0, ea1, cvec)


def _gat_scatter_pass(h, src, dst, ex):
    rpt = _NA // _NS

    @functools.partial(
        pl.kernel,
        out_type=jax.ShapeDtypeStruct((_NC, _NA, D), jnp.float32),
        mesh=_sc_mesh(),
        scratch_types=[
            pltpu.VMEM((2, _CHUNK), jnp.int32),
            pltpu.VMEM((2, _CHUNK), jnp.int32),
            pltpu.VMEM((2, _CHUNK), jnp.float32),
            pltpu.VMEM((2, _CHUNK, D), jnp.float32),
            pltpu.VMEM_SHARED((_NA, D), jnp.float32),
            pltpu.SemaphoreType.DMA,
            pltpu.SemaphoreType.DMA,
        ],
    )
    def body(h_hbm, src_hbm, dst_hbm, ex_hbm, num_out, sidx_v, didx_v, ex_v,
             hrows_v, acc_sh, sem0, sem1):
        cid = lax.axis_index("c")
        sid = lax.axis_index("s")
        w = sid * _NC + cid
        sems = (sem0, sem1)

        def zrow(r, cc):
            for k in range(D // _L):
                hrows_v[0, r, pl.ds(k * _L, _L)] = jnp.zeros((_L,),
                                                             jnp.float32)
            return cc

        lax.fori_loop(0, _CHUNK, zrow, 0)
        for b in range(rpt // _CHUNK):
            pltpu.sync_copy(hrows_v.at[0],
                            acc_sh.at[pl.ds(sid * rpt + b * _CHUNK, _CHUNK)])
        plsc.subcore_barrier()

        def fetch(c, b):
            base = c * _CHUNK
            pltpu.sync_copy(src_hbm.at[pl.ds(base, _CHUNK)], sidx_v.at[b])
            pltpu.sync_copy(dst_hbm.at[pl.ds(base, _CHUNK)], didx_v.at[b])
            pltpu.sync_copy(ex_hbm.at[pl.ds(base, _CHUNK)], ex_v.at[b])
            return pltpu.async_copy(h_hbm.at[sidx_v.at[b]], hrows_v.at[b],
                                    sems[b])

        def process(c, b):
            def scale_group(g, cc):
                sl = pl.ds(g * _L, _L)
                exg = ex_v[b, sl]
                for i in range(_L):
                    r = g * _L + i
                    exb = jnp.full((_L,), exg[i], jnp.float32)
                    for k in range(D // _L):
                        ksl = pl.ds(k * _L, _L)
                        hrows_v[b, r, ksl] = hrows_v[b, r, ksl] * exb
                return cc

            lax.fori_loop(0, _CHUNK // _L, scale_group, 0)
            pltpu.sync_copy(hrows_v.at[b], acc_sh.at[didx_v.at[b]], add=True)

        @pl.when(w < _NCHUNK)
        def _():
            fetch(w, 0)

        def step(j2, carry):
            for b in range(2):
                j = 2 * j2 + b
                c = w + _NW * j
                cn = c + _NW

                @pl.when(cn < _NCHUNK)
                def _():
                    pltpu.make_async_copy(h_hbm.at[sidx_v.at[b]],
                                          hrows_v.at[b], sems[b]).wait()
                    fetch(cn, 1 - b)
                    process(c, b)

                @pl.when((c < _NCHUNK) & (cn >= _NCHUNK))
                def _():
                    pltpu.make_async_copy(h_hbm.at[sidx_v.at[b]],
                                          hrows_v.at[b], sems[b]).wait()
                    process(c, b)

            return carry

        lax.fori_loop(0, (_JMAX + 1) // 2, step, 0)
        plsc.subcore_barrier()
        pltpu.sync_copy(acc_sh.at[pl.ds(sid * rpt, rpt)],
                        num_out.at[cid, pl.ds(sid * rpt, rpt)])

    return body(h, src, dst, ex)


_BN = 1000


def _prep_body(x_ref, w_ref, a2_ref, c_ref, cnt_ref, lsum_ref,
               h_ref, sd_ref, exl_ref, la_ref):
    h = jnp.dot(x_ref[...], w_ref[...], preferred_element_type=jnp.float32)
    sd = jnp.dot(h, a2_ref[...], preferred_element_type=jnp.float32)
    la = lsum_ref[...] / jnp.maximum(cnt_ref[...], 1.0)
    ael = jnp.dot(la, c_ref[...], preferred_element_type=jnp.float32)
    al = sd[:, 0:1] + sd[:, 1:2] + ael
    al = jnp.where(al >= 0, al, 0.2 * al)
    h_ref[...] = h
    sd_ref[...] = sd
    exl_ref[...] = jnp.exp(al)
    la_ref[...] = la


def _prep_layer(x, W, a_s, a_d, c, cnt_col, lsum):
    a2 = jnp.stack([a_s, a_d], axis=1)
    ccol = c[:, None]
    grid = (N // _BN,)
    return pl.pallas_call(
        _prep_body,
        grid=grid,
        in_specs=[
            pl.BlockSpec((_BN, D), lambda i: (i, 0)),
            pl.BlockSpec((D, D), lambda i: (0, 0)),
            pl.BlockSpec((D, 2), lambda i: (0, 0)),
            pl.BlockSpec((2, 1), lambda i: (0, 0)),
            pl.BlockSpec((_BN, 1), lambda i: (i, 0)),
            pl.BlockSpec((_BN, 2), lambda i: (i, 0)),
        ],
        out_specs=[
            pl.BlockSpec((_BN, D), lambda i: (i, 0)),
            pl.BlockSpec((_BN, 2), lambda i: (i, 0)),
            pl.BlockSpec((_BN, 1), lambda i: (i, 0)),
            pl.BlockSpec((_BN, 2), lambda i: (i, 0)),
        ],
        out_shape=[
            jax.ShapeDtypeStruct((N, D), jnp.float32),
            jax.ShapeDtypeStruct((N, 2), jnp.float32),
            jax.ShapeDtypeStruct((N, 1), jnp.float32),
            jax.ShapeDtypeStruct((N, 2), jnp.float32),
        ],
    )(x, W, a2, ccol, cnt_col, lsum)


def _finish_body(p_ref, dsum_ref, exl_ref, h_ref, b_ref, y_ref):
    num = p_ref[0] + p_ref[1] + exl_ref[...] * h_ref[...]
    den = dsum_ref[...] + exl_ref[...] + 1e-16
    y_ref[...] = jnp.maximum(num / den + b_ref[...], 0.0)


def _finish_layer(nump, dsum_col, exl, h, b):
    grid = (N // _BN,)
    return pl.pallas_call(
        _finish_body,
        grid=grid,
        in_specs=[
            pl.BlockSpec((2, _BN, D), lambda i: (0, i, 0)),
            pl.BlockSpec((_BN, 1), lambda i: (i, 0)),
            pl.BlockSpec((_BN, 1), lambda i: (i, 0)),
            pl.BlockSpec((_BN, D), lambda i: (i, 0)),
            pl.BlockSpec((1, D), lambda i: (0, 0)),
        ],
        out_specs=pl.BlockSpec((_BN, D), lambda i: (i, 0)),
        out_shape=jax.ShapeDtypeStruct((N, D), jnp.float32),
    )(nump, dsum_col, exl, h, b[None, :])


def _finish_body3(p_ref, dsum_ref, exl_ref, h_ref, b_ref, y0_ref,
                  wl2_ref, bl2_ref, y4_ref):
    num = p_ref[0] + p_ref[1] + exl_ref[...] * h_ref[...]
    den = dsum_ref[...] + exl_ref[...] + 1e-16
    y1 = jnp.maximum(num / den + b_ref[...], 0.0)
    s = y0_ref[...] + y1
    y4 = jnp.dot(s, wl2_ref[...], preferred_element_type=jnp.float32)
    y4_ref[...] = jnp.maximum(y4 + bl2_ref[...], 0.0)


def _finish_layer2(nump, dsum_col, exl, h, b, y0, Wl2, bl2):
    grid = (N // _BN,)
    return pl.pallas_call(
        _finish_body3,
        grid=grid,
        in_specs=[
            pl.BlockSpec((2, _BN, D), lambda i: (0, i, 0)),
            pl.BlockSpec((_BN, 1), lambda i: (i, 0)),
            pl.BlockSpec((_BN, 1), lambda i: (i, 0)),
            pl.BlockSpec((_BN, D), lambda i: (i, 0)),
            pl.BlockSpec((1, D), lambda i: (0, 0)),
            pl.BlockSpec((_BN, D), lambda i: (i, 0)),
            pl.BlockSpec((D, D), lambda i: (0, 0)),
            pl.BlockSpec((1, D), lambda i: (0, 0)),
        ],
        out_specs=pl.BlockSpec((_BN, D), lambda i: (i, 0)),
        out_shape=jax.ShapeDtypeStruct((N, D), jnp.float32),
    )(nump, dsum_col, exl, h, b[None, :], y0, Wl2, bl2[None, :])


def _edge_mlp_body(q_ref, ea_ref, wl3a_ref, bl3_ref, wl3b_ref, wm1_ref, bm1_ref,
                   wm2_ref, bm2_ref, wl4_ref, bl4_ref, out_ref):
    za = jnp.dot(q_ref[...], wl3a_ref[...], preferred_element_type=jnp.float32)
    eb = jnp.dot(ea_ref[...], wl3b_ref[...], preferred_element_type=jnp.float32)
    u0 = jnp.maximum(za + eb + bl3_ref[...], 0.0)
    u1 = jnp.dot(u0, wm1_ref[...], preferred_element_type=jnp.float32)
    u1 = jnp.maximum(u1 + bm1_ref[...], 0.0)
    u2 = jnp.dot(u1, wm2_ref[...], preferred_element_type=jnp.float32)
    u2 = jnp.maximum(u2 + bm2_ref[...], 0.0)
    yb = jnp.dot(u2, wl4_ref[...], preferred_element_type=jnp.float32)
    out_ref[...] = yb + bl4_ref[...]


def _edge_mlp(q, ea8, wl3a, bl3p, wl3b8, wm1p, bm1p, wm2p, bm2p, wl4p, bl4p):
    grid = (E // _BM,)
    return pl.pallas_call(
        _edge_mlp_body,
        grid=grid,
        in_specs=[
            pl.BlockSpec((_BM, D), lambda i: (i, 0)),
            pl.BlockSpec((_BM, 8), lambda i: (i, 0)),
            pl.BlockSpec((D, HP), lambda i: (0, 0)),
            pl.BlockSpec((1, HP), lambda i: (0, 0)),
            pl.BlockSpec((8, HP), lambda i: (0, 0)),
            pl.BlockSpec((HP, HP), lambda i: (0, 0)),
            pl.BlockSpec((1, HP), lambda i: (0, 0)),
            pl.BlockSpec((HP, HP), lambda i: (0, 0)),
            pl.BlockSpec((1, HP), lambda i: (0, 0)),
            pl.BlockSpec((HP, OUT), lambda i: (0, 0)),
            pl.BlockSpec((1, OUT), lambda i: (0, 0)),
        ],
        out_specs=pl.BlockSpec((_BM, OUT), lambda i: (i, 0)),
        out_shape=jax.ShapeDtypeStruct((E, OUT), jnp.float32),
    )(q, ea8, wl3a, bl3p, wl3b8, wm1p, bm1p, wm2p, bm2p, wl4p, bl4p)


def _pad2(a, r, c):
    return jnp.pad(a, ((0, r - a.shape[0]), (0, c - a.shape[1])))


def kernel(x, edge_index, edge_attr, shift, W1, a1_src, a1_dst, We1, a1_edge, b1,
           W2, a2_src, a2_dst, We2, a2_edge, b2, W_l2, b_l2, W_l3, b_l3,
           Wm1, bm1, Wm2, bm2, W_l4, b_l4):
    src = edge_index[0]
    dst = edge_index[1]
    ea0 = edge_attr[:, 0]
    ea1 = edge_attr[:, 1]
    la, _ = _loop_attr_pass(src, dst, ea0, ea1)
    las = la[0] + la[1]
    cnt_col = las[:N, None]
    lsum = jnp.stack([las[_N16:_N16 + N], las[2 * _N16:2 * _N16 + N]], axis=1)

    def gat_core(xin, W, a_s, a_d, We, a_e):
        c = We @ a_e
        h, sd, exl, _ = _prep_layer(xin, W, a_s, a_d, c, cnt_col, lsum)
        asn = sd[:, 0]
        adn = sd[:, 1]
        cvec = jnp.pad(c, (0, 14))
        ex, denp, _ = _gat_alpha_pass(asn, adn, src, dst, ea0, ea1, cvec)
        nump = _gat_scatter_pass(h, src, dst, ex)
        dsum_col = (denp[0, :N] + denp[1, :N])[:, None]
        return nump, dsum_col, exl, h

    nump1, dsum1, exl1, h1 = gat_core(x, W1, a1_src, a1_dst, We1, a1_edge)
    y0 = _finish_layer(nump1, dsum1, exl1, h1, b1)
    nump2, dsum2, exl2, h2 = gat_core(y0, W2, a2_src, a2_dst, We2, a2_edge)
    y4 = _finish_layer2(nump2, dsum2, exl2, h2, b2, y0, W_l2, b_l2)

    q = _gather_pair_sum(y4, src, dst)

    ea8 = jnp.pad(edge_attr, ((0, 0), (0, 6)))
    wl3a = jnp.pad(W_l3[:H], ((0, 0), (0, HP - (H + 2))))
    bl3p = jnp.pad(b_l3, (0, HP - (H + 2)))[None, :]
    wl3b8 = jnp.pad(W_l3[H:], ((0, 6), (0, HP - (H + 2))))
    wm1p = _pad2(Wm1, HP, HP)
    wm2p = _pad2(Wm2, HP, HP)
    wl4p = jnp.pad(W_l4, ((0, HP - (H + 2)), (0, 0)))
    bm1p = jnp.pad(bm1, (0, HP - (H + 2)))[None, :]
    bm2p = jnp.pad(bm2, (0, HP - (H + 2)))[None, :]
    bl4p = b_l4[None, :]

    return _edge_mlp(q, ea8, wl3a, bl3p, wl3b8, wm1p, bm1p, wm2p, bm2p, wl4p,
                     bl4p)


# speedup vs baseline: 1.0992x; 1.0334x over previous
import functools

import jax
import jax.numpy as jnp
from jax import lax
from jax.experimental import pallas as pl
from jax.experimental.pallas import tpu as pltpu
from jax.experimental.pallas import tpu_sc as plsc

N = 10000
E = 160000
D = 128
H = 128
HP = 144
OUT = 313

_BM = 640

_NC = 2
_NS = 16
_NW = _NC * _NS
_L = 16
_CHUNK = 128
_NCHUNK = E // _CHUNK
_JMAX = (_NCHUNK + _NW - 1) // _NW


def _sc_mesh():
    return plsc.VectorSubcoreMesh(core_axis_name="c", subcore_axis_name="s")


def _gather_pair_sum(tab, src, dst):
    @functools.partial(
        pl.kernel,
        out_type=jax.ShapeDtypeStruct((E, D), jnp.float32),
        mesh=_sc_mesh(),
        scratch_types=[
            pltpu.VMEM((2, _CHUNK), jnp.int32),
            pltpu.VMEM((2, _CHUNK), jnp.int32),
            pltpu.VMEM((2, _CHUNK, D), jnp.float32),
            pltpu.VMEM((2, _CHUNK, D), jnp.float32),
            pltpu.SemaphoreType.DMA,
            pltpu.SemaphoreType.DMA,
            pltpu.SemaphoreType.DMA,
            pltpu.SemaphoreType.DMA,
        ],
    )
    def body(tab_hbm, src_hbm, dst_hbm, out_hbm, sidx_v, didx_v, ra_v, rb_v,
             sa0, sa1, sb0, sb1):
        w = lax.axis_index("s") * _NC + lax.axis_index("c")
        sas = (sa0, sa1)
        sbs = (sb0, sb1)

        def fetch(c, b):
            base = c * _CHUNK
            pltpu.sync_copy(src_hbm.at[pl.ds(base, _CHUNK)], sidx_v.at[b])
            pltpu.sync_copy(dst_hbm.at[pl.ds(base, _CHUNK)], didx_v.at[b])
            pltpu.async_copy(tab_hbm.at[sidx_v.at[b]], ra_v.at[b], sas[b])
            pltpu.async_copy(tab_hbm.at[didx_v.at[b]], rb_v.at[b], sbs[b])

        def wait(b):
            pltpu.make_async_copy(tab_hbm.at[sidx_v.at[b]], ra_v.at[b],
                                  sas[b]).wait()
            pltpu.make_async_copy(tab_hbm.at[didx_v.at[b]], rb_v.at[b],
                                  sbs[b]).wait()

        def process(c, b):
            def add_row(r, cc):
                for k in range(D // _L):
                    sl = pl.ds(k * _L, _L)
                    ra_v[b, r, sl] = ra_v[b, r, sl] + rb_v[b, r, sl]
                return cc

            lax.fori_loop(0, _CHUNK, add_row, 0)
            pltpu.sync_copy(ra_v.at[b], out_hbm.at[pl.ds(c * _CHUNK, _CHUNK)])

        @pl.when(w < _NCHUNK)
        def _():
            fetch(w, 0)

        def step(j2, carry):
            for b in range(2):
                c = w + _NW * (2 * j2 + b)
                cn = c + _NW

                @pl.when(cn < _NCHUNK)
                def _():
                    wait(b)
                    fetch(cn, 1 - b)
                    process(c, b)

                @pl.when((c < _NCHUNK) & (cn >= _NCHUNK))
                def _():
                    wait(b)
                    process(c, b)

            return carry

        lax.fori_loop(0, (_JMAX + 1) // 2, step, 0)

    return body(tab, src, dst)


_N16 = 10240
_NA = 10240


def _loop_attr_pass(src, dst, ea0, ea1):
    cols = _N16 // _NS

    @functools.partial(
        pl.kernel,
        out_type=(jax.ShapeDtypeStruct((_NC, 3 * _N16), jnp.float32),
                  jax.ShapeDtypeStruct((_NC, _NS, 3 * _N16), jnp.float32)),
        mesh=_sc_mesh(),
        scratch_types=[
            pltpu.VMEM((3 * _N16,), jnp.float32),
            pltpu.VMEM((_CHUNK,), jnp.int32),
            pltpu.VMEM((_CHUNK,), jnp.int32),
            pltpu.VMEM((_CHUNK,), jnp.float32),
            pltpu.VMEM((_CHUNK,), jnp.float32),
        ],
        compiler_params=pltpu.CompilerParams(needs_layout_passes=False),
    )
    def body(src_hbm, dst_hbm, ea0_hbm, ea1_hbm, out_hbm, scr_hbm, tab_v,
             sidx_v, didx_v, a0_v, a1_v):
        cid = lax.axis_index("c")
        sid = lax.axis_index("s")
        w = sid * _NC + cid

        def ztab(k, cc):
            tab_v[pl.ds(k * _L, _L)] = jnp.zeros((_L,), jnp.float32)
            return cc

        lax.fori_loop(0, 3 * _N16 // _L, ztab, 0)

        def step(j, carry):
            c = w + _NW * j

            @pl.when(c < _NCHUNK)
            def _():
                base = c * _CHUNK
                pltpu.sync_copy(src_hbm.at[pl.ds(base, _CHUNK)], sidx_v)
                pltpu.sync_copy(dst_hbm.at[pl.ds(base, _CHUNK)], didx_v)
                pltpu.sync_copy(ea0_hbm.at[pl.ds(base, _CHUNK)], a0_v)
                pltpu.sync_copy(ea1_hbm.at[pl.ds(base, _CHUNK)], a1_v)
                for g in range(_CHUNK // _L):
                    sl = pl.ds(g * _L, _L)
                    si = sidx_v[sl]
                    di = didx_v[sl]
                    m = (si != di).astype(jnp.float32)
                    plsc.addupdate_scatter(tab_v, [di], m)
                    plsc.addupdate_scatter(tab_v, [di + _N16], m * a0_v[sl])
                    plsc.addupdate_scatter(tab_v, [di + 2 * _N16],
                                           m * a1_v[sl])

            return carry

        lax.fori_loop(0, _JMAX, step, 0)

        pltpu.sync_copy(tab_v, scr_hbm.at[cid, sid])
        plsc.subcore_barrier()
        for q in range(3):
            for t in range(_NS):
                pltpu.sync_copy(
                    scr_hbm.at[cid, t, pl.ds(q * _N16 + sid * cols, cols)],
                    tab_v.at[pl.ds(t * cols, cols)])

            def qred(k, cc):
                sl = pl.ds(k * _L, _L)
                v = tab_v[sl]
                for t in range(1, _NS):
                    v = v + tab_v[pl.ds(t * cols + k * _L, _L)]
                tab_v[sl] = v
                return cc

            lax.fori_loop(0, cols // _L, qred, 0)
            pltpu.sync_copy(tab_v.at[pl.ds(0, cols)],
                            out_hbm.at[cid, pl.ds(q * _N16 + sid * cols,
                                                  cols)])

    return body(src, dst, ea0, ea1)


def _gat_alpha_pass(asn, adn, src, dst, ea0, ea1, cvec):
    cols = _N16 // _NS

    @functools.partial(
        pl.kernel,
        out_type=(jax.ShapeDtypeStruct((E,), jnp.float32),
                  jax.ShapeDtypeStruct((_NC, _N16), jnp.float32),
                  jax.ShapeDtypeStruct((_NC, _NS, _N16), jnp.float32)),
        mesh=_sc_mesh(),
        scratch_types=[
            pltpu.VMEM((N,), jnp.float32),
            pltpu.VMEM((N,), jnp.float32),
            pltpu.VMEM((_N16,), jnp.float32),
            pltpu.VMEM((_CHUNK,), jnp.int32),
            pltpu.VMEM((_CHUNK,), jnp.int32),
            pltpu.VMEM((_CHUNK,), jnp.float32),
            pltpu.VMEM((_CHUNK,), jnp.float32),
            pltpu.VMEM((_CHUNK,), jnp.float32),
            pltpu.VMEM((16,), jnp.float32),
        ],
        compiler_params=pltpu.CompilerParams(needs_layout_passes=False),
    )
    def body(asn_hbm, adn_hbm, src_hbm, dst_hbm, ea0_hbm, ea1_hbm, cvec_hbm,
             ex_out, den_out, den_scr, asn_v, adn_v, den_v, sidx_v, didx_v,
             a0_v, a1_v, ex_v, cv_v):
        cid = lax.axis_index("c")
        sid = lax.axis_index("s")
        w = sid * _NC + cid

        pltpu.sync_copy(asn_hbm, asn_v)
        pltpu.sync_copy(adn_hbm, adn_v)
        pltpu.sync_copy(cvec_hbm, cv_v)

        def zden(k, cc):
            den_v[pl.ds(k * _L, _L)] = jnp.zeros((_L,), jnp.float32)
            return cc

        lax.fori_loop(0, _N16 // _L, zden, 0)
        cvv = cv_v[pl.ds(0, _L)]
        c0 = cvv[0]
        c1 = cvv[1]

        def step(j, carry):
            c = w + _NW * j

            @pl.when(c < _NCHUNK)
            def _():
                base = c * _CHUNK
                pltpu.sync_copy(src_hbm.at[pl.ds(base, _CHUNK)], sidx_v)
                pltpu.sync_copy(dst_hbm.at[pl.ds(base, _CHUNK)], didx_v)
                pltpu.sync_copy(ea0_hbm.at[pl.ds(base, _CHUNK)], a0_v)
                pltpu.sync_copy(ea1_hbm.at[pl.ds(base, _CHUNK)], a1_v)
                for g in range(_CHUNK // _L):
                    sl = pl.ds(g * _L, _L)
                    si = sidx_v[sl]
                    di = didx_v[sl]
                    av = plsc.load_gather(asn_v, [si])
                    bv = plsc.load_gather(adn_v, [di])
                    al = av + bv + a0_v[sl] * c0 + a1_v[sl] * c1
                    al = jnp.where(al >= 0, al, 0.2 * al)
                    exg = jnp.where(si != di, jnp.exp(al), 0.0)
                    ex_v[sl] = exg
                    plsc.addupdate_scatter(den_v, [di], exg)
                pltpu.sync_copy(ex_v, ex_out.at[pl.ds(base, _CHUNK)])

            return carry

        lax.fori_loop(0, _JMAX, step, 0)

        pltpu.sync_copy(den_v, den_scr.at[cid, sid])
        plsc.subcore_barrier()
        for t in range(_NS):
            pltpu.sync_copy(den_scr.at[cid, t, pl.ds(sid * cols, cols)],
                            den_v.at[pl.ds(t * cols, cols)])

        def dred(k, cc):
            sl = pl.ds(k * _L, _L)
            v = den_v[sl]
            for t in range(1, _NS):
                v = v + den_v[pl.ds(t * cols + k * _L, _L)]
            den_v[sl] = v
            return cc

        lax.fori_loop(0, cols // _L, dred, 0)
        pltpu.sync_copy(den_v.at[pl.ds(0, cols)],
                        den_out.at[cid, pl.ds(sid * cols, cols)])

    return body(asn, adn, src, dst, ea0, ea1, cvec)


def _gat_scatter_pass(h, src, dst, ex):
    rpt = _NA // _NS

    @functools.partial(
        pl.kernel,
        out_type=jax.ShapeDtypeStruct((_NC, _NA, D), jnp.float32),
        mesh=_sc_mesh(),
        scratch_types=[
            pltpu.VMEM((2, _CHUNK), jnp.int32),
            pltpu.VMEM((2, _CHUNK), jnp.int32),
            pltpu.VMEM((2, _CHUNK), jnp.float32),
            pltpu.VMEM((2, _CHUNK, D), jnp.float32),
            pltpu.VMEM_SHARED((_NA, D), jnp.float32),
            pltpu.SemaphoreType.DMA,
            pltpu.SemaphoreType.DMA,
        ],
    )
    def body(h_hbm, src_hbm, dst_hbm, ex_hbm, num_out, sidx_v, didx_v, ex_v,
             hrows_v, acc_sh, sem0, sem1):
        cid = lax.axis_index("c")
        sid = lax.axis_index("s")
        w = sid * _NC + cid
        sems = (sem0, sem1)

        def zrow(r, cc):
            for k in range(D // _L):
                hrows_v[0, r, pl.ds(k * _L, _L)] = jnp.zeros((_L,),
                                                             jnp.float32)
            return cc

        lax.fori_loop(0, _CHUNK, zrow, 0)
        for b in range(rpt // _CHUNK):
            pltpu.sync_copy(hrows_v.at[0],
                            acc_sh.at[pl.ds(sid * rpt + b * _CHUNK, _CHUNK)])
        plsc.subcore_barrier()

        def fetch(c, b):
            base = c * _CHUNK
            pltpu.sync_copy(src_hbm.at[pl.ds(base, _CHUNK)], sidx_v.at[b])
            pltpu.sync_copy(dst_hbm.at[pl.ds(base, _CHUNK)], didx_v.at[b])
            pltpu.sync_copy(ex_hbm.at[pl.ds(base, _CHUNK)], ex_v.at[b])
            return pltpu.async_copy(h_hbm.at[sidx_v.at[b]], hrows_v.at[b],
                                    sems[b])

        def process(c, b):
            def scale_group(g, cc):
                sl = pl.ds(g * _L, _L)
                exg = ex_v[b, sl]
                for i in range(_L):
                    r = g * _L + i
                    exb = jnp.full((_L,), exg[i], jnp.float32)
                    for k in range(D // _L):
                        ksl = pl.ds(k * _L, _L)
                        hrows_v[b, r, ksl] = hrows_v[b, r, ksl] * exb
                return cc

            lax.fori_loop(0, _CHUNK // _L, scale_group, 0)
            pltpu.sync_copy(hrows_v.at[b], acc_sh.at[didx_v.at[b]], add=True)

        @pl.when(w < _NCHUNK)
        def _():
            fetch(w, 0)

        def step(j2, carry):
            for b in range(2):
                j = 2 * j2 + b
                c = w + _NW * j
                cn = c + _NW

                @pl.when(cn < _NCHUNK)
                def _():
                    pltpu.make_async_copy(h_hbm.at[sidx_v.at[b]],
                                          hrows_v.at[b], sems[b]).wait()
                    fetch(cn, 1 - b)
                    process(c, b)

                @pl.when((c < _NCHUNK) & (cn >= _NCHUNK))
                def _():
                    pltpu.make_async_copy(h_hbm.at[sidx_v.at[b]],
                                          hrows_v.at[b], sems[b]).wait()
                    process(c, b)

            return carry

        lax.fori_loop(0, (_JMAX + 1) // 2, step, 0)
        plsc.subcore_barrier()
        pltpu.sync_copy(acc_sh.at[pl.ds(sid * rpt, rpt)],
                        num_out.at[cid, pl.ds(sid * rpt, rpt)])

    return body(h, src, dst, ex)


_BN = 1000


def _prep_body(x_ref, w_ref, a2_ref, c_ref, cnt_ref, lsum_ref,
               h_ref, sd_ref, exl_ref, la_ref):
    h = jnp.dot(x_ref[...], w_ref[...], preferred_element_type=jnp.float32)
    sd = jnp.dot(h, a2_ref[...], preferred_element_type=jnp.float32)
    la = lsum_ref[...] / jnp.maximum(cnt_ref[...], 1.0)
    ael = jnp.dot(la, c_ref[...], preferred_element_type=jnp.float32)
    al = sd[:, 0:1] + sd[:, 1:2] + ael
    al = jnp.where(al >= 0, al, 0.2 * al)
    h_ref[...] = h
    sd_ref[...] = sd
    exl_ref[...] = jnp.exp(al)
    la_ref[...] = la


def _prep_layer(x, W, a_s, a_d, c, cnt_col, lsum):
    a2 = jnp.stack([a_s, a_d], axis=1)
    ccol = c[:, None]
    grid = (N // _BN,)
    return pl.pallas_call(
        _prep_body,
        grid=grid,
        in_specs=[
            pl.BlockSpec((_BN, D), lambda i: (i, 0)),
            pl.BlockSpec((D, D), lambda i: (0, 0)),
            pl.BlockSpec((D, 2), lambda i: (0, 0)),
            pl.BlockSpec((2, 1), lambda i: (0, 0)),
            pl.BlockSpec((_BN, 1), lambda i: (i, 0)),
            pl.BlockSpec((_BN, 2), lambda i: (i, 0)),
        ],
        out_specs=[
            pl.BlockSpec((_BN, D), lambda i: (i, 0)),
            pl.BlockSpec((_BN, 2), lambda i: (i, 0)),
            pl.BlockSpec((_BN, 1), lambda i: (i, 0)),
            pl.BlockSpec((_BN, 2), lambda i: (i, 0)),
        ],
        out_shape=[
            jax.ShapeDtypeStruct((N, D), jnp.float32),
            jax.ShapeDtypeStruct((N, 2), jnp.float32),
            jax.ShapeDtypeStruct((N, 1), jnp.float32),
            jax.ShapeDtypeStruct((N, 2), jnp.float32),
        ],
    )(x, W, a2, ccol, cnt_col, lsum)


def _finish_body(p_ref, dsum_ref, exl_ref, h_ref, b_ref, y_ref):
    num = p_ref[0] + p_ref[1] + exl_ref[...] * h_ref[...]
    den = dsum_ref[...] + exl_ref[...] + 1e-16
    y_ref[...] = jnp.maximum(num / den + b_ref[...], 0.0)


def _finish_layer(nump, dsum_col, exl, h, b):
    grid = (N // _BN,)
    return pl.pallas_call(
        _finish_body,
        grid=grid,
        in_specs=[
            pl.BlockSpec((2, _BN, D), lambda i: (0, i, 0)),
            pl.BlockSpec((_BN, 1), lambda i: (i, 0)),
            pl.BlockSpec((_BN, 1), lambda i: (i, 0)),
            pl.BlockSpec((_BN, D), lambda i: (i, 0)),
            pl.BlockSpec((1, D), lambda i: (0, 0)),
        ],
        out_specs=pl.BlockSpec((_BN, D), lambda i: (i, 0)),
        out_shape=jax.ShapeDtypeStruct((N, D), jnp.float32),
    )(nump, dsum_col, exl, h, b[None, :])


def _finish_body3(p_ref, dsum_ref, exl_ref, h_ref, b_ref, y0_ref,
                  wl2_ref, bl2_ref, y4_ref):
    num = p_ref[0] + p_ref[1] + exl_ref[...] * h_ref[...]
    den = dsum_ref[...] + exl_ref[...] + 1e-16
    y1 = jnp.maximum(num / den + b_ref[...], 0.0)
    s = y0_ref[...] + y1
    y4 = jnp.dot(s, wl2_ref[...], preferred_element_type=jnp.float32)
    y4_ref[...] = jnp.maximum(y4 + bl2_ref[...], 0.0)


def _finish_layer2(nump, dsum_col, exl, h, b, y0, Wl2, bl2):
    grid = (N // _BN,)
    return pl.pallas_call(
        _finish_body3,
        grid=grid,
        in_specs=[
            pl.BlockSpec((2, _BN, D), lambda i: (0, i, 0)),
            pl.BlockSpec((_BN, 1), lambda i: (i, 0)),
            pl.BlockSpec((_BN, 1), lambda i: (i, 0)),
            pl.BlockSpec((_BN, D), lambda i: (i, 0)),
            pl.BlockSpec((1, D), lambda i: (0, 0)),
            pl.BlockSpec((_BN, D), lambda i: (i, 0)),
            pl.BlockSpec((D, D), lambda i: (0, 0)),
            pl.BlockSpec((1, D), lambda i: (0, 0)),
        ],
        out_specs=pl.BlockSpec((_BN, D), lambda i: (i, 0)),
        out_shape=jax.ShapeDtypeStruct((N, D), jnp.float32),
    )(nump, dsum_col, exl, h, b[None, :], y0, Wl2, bl2[None, :])


def _edge_mlp_body(q_ref, ea_ref, wl3a_ref, bl3_ref, wl3b_ref, wm1_ref, bm1_ref,
                   wm2_ref, bm2_ref, wl4_ref, bl4_ref, out_ref):
    za = jnp.dot(q_ref[...], wl3a_ref[...], preferred_element_type=jnp.float32)
    eb = jnp.dot(ea_ref[...], wl3b_ref[...], preferred_element_type=jnp.float32)
    u0 = jnp.maximum(za + eb + bl3_ref[...], 0.0)
    u1 = jnp.dot(u0, wm1_ref[...], preferred_element_type=jnp.float32)
    u1 = jnp.maximum(u1 + bm1_ref[...], 0.0)
    u2 = jnp.dot(u1, wm2_ref[...], preferred_element_type=jnp.float32)
    u2 = jnp.maximum(u2 + bm2_ref[...], 0.0)
    yb = jnp.dot(u2, wl4_ref[...], preferred_element_type=jnp.float32)
    out_ref[...] = yb + bl4_ref[...]


def _edge_mlp(q, ea8, wl3a, bl3p, wl3b8, wm1p, bm1p, wm2p, bm2p, wl4p, bl4p):
    grid = (E // _BM,)
    return pl.pallas_call(
        _edge_mlp_body,
        grid=grid,
        in_specs=[
            pl.BlockSpec((_BM, D), lambda i: (i, 0)),
            pl.BlockSpec((_BM, 8), lambda i: (i, 0)),
            pl.BlockSpec((D, HP), lambda i: (0, 0)),
            pl.BlockSpec((1, HP), lambda i: (0, 0)),
            pl.BlockSpec((8, HP), lambda i: (0, 0)),
            pl.BlockSpec((HP, HP), lambda i: (0, 0)),
            pl.BlockSpec((1, HP), lambda i: (0, 0)),
            pl.BlockSpec((HP, HP), lambda i: (0, 0)),
            pl.BlockSpec((1, HP), lambda i: (0, 0)),
            pl.BlockSpec((HP, OUT), lambda i: (0, 0)),
            pl.BlockSpec((1, OUT), lambda i: (0, 0)),
        ],
        out_specs=pl.BlockSpec((_BM, OUT), lambda i: (i, 0)),
        out_shape=jax.ShapeDtypeStruct((E, OUT), jnp.float32),
    )(q, ea8, wl3a, bl3p, wl3b8, wm1p, bm1p, wm2p, bm2p, wl4p, bl4p)


def _pad2(a, r, c):
    return jnp.pad(a, ((0, r - a.shape[0]), (0, c - a.shape[1])))


def kernel(x, edge_index, edge_attr, shift, W1, a1_src, a1_dst, We1, a1_edge, b1,
           W2, a2_src, a2_dst, We2, a2_edge, b2, W_l2, b_l2, W_l3, b_l3,
           Wm1, bm1, Wm2, bm2, W_l4, b_l4):
    src = edge_index[0]
    dst = edge_index[1]
    ea0 = edge_attr[:, 0]
    ea1 = edge_attr[:, 1]
    la, _ = _loop_attr_pass(src, dst, ea0, ea1)
    las = la[0] + la[1]
    cnt_col = las[:N, None]
    lsum = jnp.stack([las[_N16:_N16 + N], las[2 * _N16:2 * _N16 + N]], axis=1)

    def gat_core(xin, W, a_s, a_d, We, a_e):
        c = We @ a_e
        h, sd, exl, _ = _prep_layer(xin, W, a_s, a_d, c, cnt_col, lsum)
        asn = sd[:, 0]
        adn = sd[:, 1]
        cvec = jnp.pad(c, (0, 14))
        ex, denp, _ = _gat_alpha_pass(asn, adn, src, dst, ea0, ea1, cvec)
        nump = _gat_scatter_pass(h, src, dst, ex)
        dsum_col = (denp[0, :N] + denp[1, :N])[:, None]
        return nump, dsum_col, exl, h

    nump1, dsum1, exl1, h1 = gat_core(x, W1, a1_src, a1_dst, We1, a1_edge)
    y0 = _finish_layer(nump1, dsum1, exl1, h1, b1)
    nump2, dsum2, exl2, h2 = gat_core(y0, W2, a2_src, a2_dst, We2, a2_edge)
    y4 = _finish_layer2(nump2, dsum2, exl2, h2, b2, y0, W_l2, b_l2)

    q = _gather_pair_sum(y4, src, dst)

    ea8 = jnp.pad(edge_attr, ((0, 0), (0, 6)))
    wl3a = jnp.pad(W_l3[:H], ((0, 0), (0, HP - (H + 2))))
    bl3p = jnp.pad(b_l3, (0, HP - (H + 2)))[None, :]
    wl3b8 = jnp.pad(W_l3[H:], ((0, 6), (0, HP - (H + 2))))
    wm1p = _pad2(Wm1, HP, HP)
    wm2p = _pad2(Wm2, HP, HP)
    wl4p = jnp.pad(W_l4, ((0, HP - (H + 2)), (0, 0)))
    bm1p = jnp.pad(bm1, (0, HP - (H + 2)))[None, :]
    bm2p = jnp.pad(bm2, (0, HP - (H + 2)))[None, :]
    bl4p = b_l4[None, :]

    return _edge_mlp(q, ea8, wl3a, bl3p, wl3b8, wm1p, bm1p, wm2p, bm2p, wl4p,
                     bl4p)
